# baseline (speedup 1.0000x reference)
import functools

import jax
import jax.numpy as jnp
from jax import lax
from jax.experimental import pallas as pl
from jax.experimental.pallas import tpu as pltpu
from jax.experimental.pallas import tpu_sc as plsc

_info = plsc.get_sparse_core_info()
_NC, _NS = _info.num_cores, _info.num_subcores
_NW = _NC * _NS


@functools.lru_cache(maxsize=None)
def _make_gather(V, D, N):
    n_per_w = N // _NW
    C = 800
    n_chunks = n_per_w // C

    mesh = plsc.VectorSubcoreMesh(core_axis_name="c", subcore_axis_name="s")

    @functools.partial(
        pl.kernel,
        mesh=mesh,
        out_type=jax.ShapeDtypeStruct((N, D), jnp.float32),
        scratch_types=[
            pltpu.VMEM((n_per_w,), jnp.int32),
            pltpu.VMEM((C, D), jnp.float32),
            pltpu.SemaphoreType.DMA,
        ],
        compiler_params=pltpu.CompilerParams(use_tc_tiling_on_sc=False),
    )
    def k(table_hbm, idx_hbm, out_hbm, idx_v, rows_v, gsem):
        wid = lax.axis_index("s") * _NC + lax.axis_index("c")
        base = pl.multiple_of(wid * n_per_w, 8)
        pltpu.sync_copy(idx_hbm.at[pl.ds(base, n_per_w)], idx_v)

        def body(i, carry):
            off = pl.multiple_of(i * C, 8)
            pltpu.async_copy(
                table_hbm.at[idx_v.at[pl.ds(off, C)]], rows_v, gsem
            ).wait()
            pltpu.sync_copy(rows_v, out_hbm.at[pl.ds(base + off, C)])
            return carry

        lax.fori_loop(0, n_chunks, body, 0)

    return k


def kernel(tokens, weights):
    B, H = tokens.shape
    V, D = weights.shape
    N = B * H
    idx = tokens.reshape(N).astype(jnp.int32)
    out = _make_gather(V, D, N)(weights, idx)
    return out.reshape(B, H, D)


# speedup vs baseline: 1.0085x; 1.0085x over previous
import functools

import jax
import jax.numpy as jnp
from jax import lax
from jax.experimental import pallas as pl
from jax.experimental.pallas import tpu as pltpu
from jax.experimental.pallas import tpu_sc as plsc

_info = plsc.get_sparse_core_info()
_NC, _NS = _info.num_cores, _info.num_subcores
_NW = _NC * _NS


@functools.lru_cache(maxsize=None)
def _make_gather(V, D, N):
    n_per_w = N // _NW
    C = 800
    n_chunks = n_per_w // C

    mesh = plsc.VectorSubcoreMesh(core_axis_name="c", subcore_axis_name="s")

    @functools.partial(
        pl.kernel,
        mesh=mesh,
        out_type=jax.ShapeDtypeStruct((N, D), jnp.float32),
        scratch_types=[
            pltpu.VMEM((n_per_w,), jnp.int32),
            pltpu.VMEM((2, C, D), jnp.float32),
            pltpu.SemaphoreType.DMA,
            pltpu.SemaphoreType.DMA,
            pltpu.SemaphoreType.DMA,
            pltpu.SemaphoreType.DMA,
        ],
        compiler_params=pltpu.CompilerParams(use_tc_tiling_on_sc=False),
    )
    def k(table_hbm, idx_hbm, out_hbm, idx_v, rows_v, gs0, gs1, os0, os1):
        wid = lax.axis_index("s") * _NC + lax.axis_index("c")
        base = pl.multiple_of(wid * n_per_w, 8)
        pltpu.sync_copy(idx_hbm.at[pl.ds(base, n_per_w)], idx_v)
        gsem = (gs0, gs1)
        osem = (os0, os1)

        def start_gather(i, b):
            return pltpu.async_copy(
                table_hbm.at[idx_v.at[pl.ds(i * C, C)]], rows_v.at[b], gsem[b]
            )

        def start_out(i, b):
            return pltpu.async_copy(
                rows_v.at[b], out_hbm.at[pl.ds(base + i * C, C)], osem[b]
            )

        g = [start_gather(0, 0), None]
        o = [None, None]
        for i in range(n_chunks):
            b = i % 2
            if i + 1 < n_chunks:
                if o[1 - b] is not None:
                    o[1 - b].wait()
                g[1 - b] = start_gather(i + 1, 1 - b)
            g[b].wait()
            o[b] = start_out(i, b)
        o[(n_chunks - 2) % 2].wait()
        o[(n_chunks - 1) % 2].wait()

    return k


def kernel(tokens, weights):
    B, H = tokens.shape
    V, D = weights.shape
    N = B * H
    idx = tokens.reshape(N).astype(jnp.int32)
    out = _make_gather(V, D, N)(weights, idx)
    return out.reshape(B, H, D)
